# speedup vs baseline: 3.0719x; 3.0719x over previous
import functools

import jax
import jax.numpy as jnp
from jax import lax
from jax.experimental import pallas as pl
from jax.experimental.pallas import tpu as pltpu
from jax.experimental.pallas import tpu_sc as plsc

_B, _S, _D, _L = 4, 8192, 128, 16


@functools.cache
def _make_kernel():
    info = plsc.get_sparse_core_info()
    nc, ns = info.num_cores, info.num_subcores
    nw = nc * ns
    p_per_w = _S // nw
    mesh = plsc.VectorSubcoreMesh(core_axis_name="c", subcore_axis_name="s")

    @functools.partial(
        pl.kernel,
        mesh=mesh,
        out_type=jax.ShapeDtypeStruct((_B * _S, _D), jnp.float32),
        scratch_types=[
            pltpu.VMEM((p_per_w,), jnp.int32),
            pltpu.VMEM((p_per_w, _D), jnp.float32),
            pltpu.VMEM((p_per_w, _D), jnp.float32),
            pltpu.SemaphoreType.DMA,
        ],
    )
    def k(idx_hbm, wemb_hbm, pemb_hbm, out_hbm, idx_v, pos_v, rows_v, sem):
        wid = lax.axis_index("s") * nc + lax.axis_index("c")
        pbase = wid * p_per_w
        pltpu.sync_copy(pemb_hbm.at[pl.ds(pbase, p_per_w)], pos_v)
        for b in range(_B):
            base = b * _S + pbase
            pltpu.sync_copy(idx_hbm.at[pl.ds(base, p_per_w)], idx_v)
            pltpu.async_copy(wemb_hbm.at[idx_v], rows_v, sem).wait()

            def body(r, carry):
                for c in range(_D // _L):
                    sl = pl.ds(c * _L, _L)
                    rows_v[r, sl] = rows_v[r, sl] + pos_v[r, sl]
                return carry

            lax.fori_loop(0, p_per_w, body, 0)
            pltpu.sync_copy(rows_v, out_hbm.at[pl.ds(base, p_per_w)])

    return k


def kernel(input_ids, word_embeddings, position_embeddings):
    idx = input_ids.reshape(-1).astype(jnp.int32)
    out = _make_kernel()(idx, word_embeddings, position_embeddings)
    return out.reshape(_B, _S, _D)


# speedup vs baseline: 3.6592x; 1.1912x over previous
import functools

import jax
import jax.numpy as jnp
from jax import lax
from jax.experimental import pallas as pl
from jax.experimental.pallas import tpu as pltpu
from jax.experimental.pallas import tpu_sc as plsc

_B, _S, _D, _L = 4, 8192, 128, 16


@functools.cache
def _make_kernel():
    info = plsc.get_sparse_core_info()
    nc, ns = info.num_cores, info.num_subcores
    nw = nc * ns
    p_per_w = _S // nw
    mesh = plsc.VectorSubcoreMesh(core_axis_name="c", subcore_axis_name="s")

    @functools.partial(
        pl.kernel,
        mesh=mesh,
        out_type=jax.ShapeDtypeStruct((_B * _S, _D), jnp.float32),
        scratch_types=[
            *[pltpu.VMEM((p_per_w,), jnp.int32) for _ in range(_B)],
            pltpu.VMEM((p_per_w, _D), jnp.float32),
            pltpu.VMEM((p_per_w, _D), jnp.float32),
            pltpu.VMEM((p_per_w, _D), jnp.float32),
            pltpu.SemaphoreType.DMA,
            pltpu.SemaphoreType.DMA,
            pltpu.SemaphoreType.DMA,
            pltpu.SemaphoreType.DMA,
        ],
    )
    def k(idx_hbm, wemb_hbm, pemb_hbm, out_hbm,
          i0, i1, i2, i3, pos_v, r0, r1, gs0, gs1, os0, os1):
        wid = lax.axis_index("s") * nc + lax.axis_index("c")
        pbase = wid * p_per_w
        idxs = (i0, i1, i2, i3)
        rows = (r0, r1)
        gsem = (gs0, gs1)
        osem = (os0, os1)

        pltpu.sync_copy(idx_hbm.at[pl.ds(pbase, p_per_w)], idxs[0])
        gcur = [pltpu.async_copy(wemb_hbm.at[idxs[0]], rows[0], gsem[0]), None]
        pltpu.sync_copy(idx_hbm.at[pl.ds(_S + pbase, p_per_w)], idxs[1])
        gcur[1] = pltpu.async_copy(wemb_hbm.at[idxs[1]], rows[1], gsem[1])
        for b in range(2, _B):
            pltpu.sync_copy(idx_hbm.at[pl.ds(b * _S + pbase, p_per_w)], idxs[b])
        pltpu.sync_copy(pemb_hbm.at[pl.ds(pbase, p_per_w)], pos_v)

        ocur = [None, None]
        for b in range(_B):
            buf = b % 2
            gcur[buf].wait()
            rbuf = rows[buf]

            @plsc.parallel_loop(0, p_per_w, unroll=4)
            def add_body(r, rbuf=rbuf):
                for c in range(_D // _L):
                    sl = pl.ds(c * _L, _L)
                    rbuf[r, sl] = rbuf[r, sl] + pos_v[r, sl]

            ocur[buf] = pltpu.async_copy(
                rbuf, out_hbm.at[pl.ds(b * _S + pbase, p_per_w)], osem[buf])
            if b + 2 < _B:
                ocur[buf].wait()
                gcur[buf] = pltpu.async_copy(
                    wemb_hbm.at[idxs[b + 2]], rows[buf], gsem[buf])
        ocur[0].wait()
        ocur[1].wait()

    return k


def kernel(input_ids, word_embeddings, position_embeddings):
    idx = input_ids.reshape(-1).astype(jnp.int32)
    out = _make_kernel()(idx, word_embeddings, position_embeddings)
    return out.reshape(_B, _S, _D)


# speedup vs baseline: 3.6731x; 1.0038x over previous
import functools

import jax
import jax.numpy as jnp
from jax import lax
from jax.experimental import pallas as pl
from jax.experimental.pallas import tpu as pltpu
from jax.experimental.pallas import tpu_sc as plsc

_B, _S, _D, _L = 4, 8192, 128, 16


@functools.cache
def _make_kernel():
    info = plsc.get_sparse_core_info()
    nc, ns = info.num_cores, info.num_subcores
    nw = nc * ns
    p_per_w = _S // nw
    mesh = plsc.VectorSubcoreMesh(core_axis_name="c", subcore_axis_name="s")

    @functools.partial(
        pl.kernel,
        mesh=mesh,
        out_type=jax.ShapeDtypeStruct((_B * _S, _D), jnp.float32),
        scratch_types=[
            *[pltpu.VMEM((p_per_w,), jnp.int32) for _ in range(_B)],
            pltpu.VMEM((p_per_w, _D), jnp.float32),
            pltpu.VMEM((p_per_w, _D), jnp.float32),
            pltpu.VMEM((p_per_w, _D), jnp.float32),
            pltpu.SemaphoreType.DMA,
            pltpu.SemaphoreType.DMA,
            pltpu.SemaphoreType.DMA,
            pltpu.SemaphoreType.DMA,
        ],
    )
    def k(idx_hbm, wemb_hbm, pemb_hbm, out_hbm,
          i0, i1, i2, i3, pos_v, r0, r1, gs0, gs1, os0, os1):
        wid = lax.axis_index("s") * nc + lax.axis_index("c")
        pbase = wid * p_per_w
        idxs = (i0, i1, i2, i3)
        rows = (r0, r1)
        gsem = (gs0, gs1)
        osem = (os0, os1)

        pltpu.sync_copy(idx_hbm.at[pl.ds(pbase, p_per_w)], idxs[0])
        gcur = [pltpu.async_copy(wemb_hbm.at[idxs[0]], rows[0], gsem[0]), None]
        pltpu.sync_copy(idx_hbm.at[pl.ds(_S + pbase, p_per_w)], idxs[1])
        gcur[1] = pltpu.async_copy(wemb_hbm.at[idxs[1]], rows[1], gsem[1])
        for b in range(2, _B):
            pltpu.sync_copy(idx_hbm.at[pl.ds(b * _S + pbase, p_per_w)], idxs[b])
        pltpu.sync_copy(pemb_hbm.at[pl.ds(pbase, p_per_w)], pos_v)

        ocur = [None, None]
        for b in range(_B):
            buf = b % 2
            gcur[buf].wait()
            rbuf = rows[buf]

            @plsc.parallel_loop(0, p_per_w, unroll=4)
            def add_body(r, rbuf=rbuf):
                for c in range(_D // _L):
                    sl = pl.ds(c * _L, _L)
                    plsc.addupdate(rbuf.at[r, sl], pos_v[r, sl])

            ocur[buf] = pltpu.async_copy(
                rbuf, out_hbm.at[pl.ds(b * _S + pbase, p_per_w)], osem[buf])
            if b + 2 < _B:
                ocur[buf].wait()
                gcur[buf] = pltpu.async_copy(
                    wemb_hbm.at[idxs[b + 2]], rows[buf], gsem[buf])
        ocur[0].wait()
        ocur[1].wait()

    return k


def kernel(input_ids, word_embeddings, position_embeddings):
    idx = input_ids.reshape(-1).astype(jnp.int32)
    out = _make_kernel()(idx, word_embeddings, position_embeddings)
    return out.reshape(_B, _S, _D)


# speedup vs baseline: 3.6775x; 1.0012x over previous
import functools

import jax
import jax.numpy as jnp
from jax import lax
from jax.experimental import pallas as pl
from jax.experimental.pallas import tpu as pltpu
from jax.experimental.pallas import tpu_sc as plsc

_B, _S, _D, _L = 4, 8192, 128, 16


@functools.cache
def _make_kernel():
    info = plsc.get_sparse_core_info()
    nc, ns = info.num_cores, info.num_subcores
    nw = nc * ns
    p_per_w = _S // nw
    mesh = plsc.VectorSubcoreMesh(core_axis_name="c", subcore_axis_name="s")

    @functools.partial(
        pl.kernel,
        mesh=mesh,
        out_type=jax.ShapeDtypeStruct((_B, _S, _D), jnp.float32),
        scratch_types=[
            *[pltpu.VMEM((p_per_w,), jnp.int32) for _ in range(_B)],
            pltpu.VMEM((p_per_w, _D), jnp.float32),
            pltpu.VMEM((p_per_w, _D), jnp.float32),
            pltpu.VMEM((p_per_w, _D), jnp.float32),
            pltpu.SemaphoreType.DMA,
            pltpu.SemaphoreType.DMA,
            pltpu.SemaphoreType.DMA,
            pltpu.SemaphoreType.DMA,
        ],
    )
    def k(idx_hbm, wemb_hbm, pemb_hbm, out_hbm,
          i0, i1, i2, i3, pos_v, r0, r1, gs0, gs1, os0, os1):
        wid = lax.axis_index("s") * nc + lax.axis_index("c")
        pbase = wid * p_per_w
        idxs = (i0, i1, i2, i3)
        rows = (r0, r1)
        gsem = (gs0, gs1)
        osem = (os0, os1)

        pltpu.sync_copy(idx_hbm.at[0, pl.ds(pbase, p_per_w)], idxs[0])
        gcur = [pltpu.async_copy(wemb_hbm.at[idxs[0]], rows[0], gsem[0]), None]
        pltpu.sync_copy(idx_hbm.at[1, pl.ds(pbase, p_per_w)], idxs[1])
        gcur[1] = pltpu.async_copy(wemb_hbm.at[idxs[1]], rows[1], gsem[1])
        for b in range(2, _B):
            pltpu.sync_copy(idx_hbm.at[b, pl.ds(pbase, p_per_w)], idxs[b])
        pltpu.sync_copy(pemb_hbm.at[pl.ds(pbase, p_per_w)], pos_v)

        ocur = [None, None]
        for b in range(_B):
            buf = b % 2
            gcur[buf].wait()
            rbuf = rows[buf]

            @plsc.parallel_loop(0, p_per_w, unroll=4)
            def add_body(r, rbuf=rbuf):
                for c in range(_D // _L):
                    sl = pl.ds(c * _L, _L)
                    plsc.addupdate(rbuf.at[r, sl], pos_v[r, sl])

            ocur[buf] = pltpu.async_copy(
                rbuf, out_hbm.at[b, pl.ds(pbase, p_per_w)], osem[buf])
            if b + 2 < _B:
                ocur[buf].wait()
                gcur[buf] = pltpu.async_copy(
                    wemb_hbm.at[idxs[b + 2]], rows[buf], gsem[buf])
        ocur[0].wait()
        ocur[1].wait()

    return k


def kernel(input_ids, word_embeddings, position_embeddings):
    if input_ids.dtype != jnp.int32:
        input_ids = input_ids.astype(jnp.int32)
    return _make_kernel()(input_ids, word_embeddings, position_embeddings)


# speedup vs baseline: 3.7489x; 1.0194x over previous
import functools

import jax
import jax.numpy as jnp
from jax import lax
from jax.experimental import pallas as pl
from jax.experimental.pallas import tpu as pltpu
from jax.experimental.pallas import tpu_sc as plsc

_B, _S, _D, _L = 4, 8192, 128, 16
_C = 128
_DEPTH = 4


@functools.cache
def _make_kernel():
    info = plsc.get_sparse_core_info()
    nc, ns = info.num_cores, info.num_subcores
    nw = nc * ns
    p_per_w = _S // nw
    n_items = _B * p_per_w // _C
    chunks_per_b = p_per_w // _C
    mesh = plsc.VectorSubcoreMesh(core_axis_name="c", subcore_axis_name="s")

    @functools.partial(
        pl.kernel,
        mesh=mesh,
        out_type=jax.ShapeDtypeStruct((_B, _S, _D), jnp.float32),
        scratch_types=[
            pltpu.VMEM((_B, p_per_w), jnp.int32),
            pltpu.VMEM((p_per_w, _D), jnp.float32),
            *[pltpu.VMEM((_C, _D), jnp.float32) for _ in range(_DEPTH)],
            pltpu.SemaphoreType.DMA,
            pltpu.SemaphoreType.DMA,
            *[pltpu.SemaphoreType.DMA for _ in range(_DEPTH)],
            *[pltpu.SemaphoreType.DMA for _ in range(_DEPTH)],
        ],
    )
    def k(idx_hbm, wemb_hbm, pemb_hbm, out_hbm,
          idx_v, pos_v, r0, r1, r2, r3, isem, psem,
          gs0, gs1, gs2, gs3, os0, os1, os2, os3):
        wid = lax.axis_index("s") * nc + lax.axis_index("c")
        pbase = wid * p_per_w
        rows = (r0, r1, r2, r3)
        gsem = (gs0, gs1, gs2, gs3)
        osem = (os0, os1, os2, os3)

        def item_idx(j):
            b, h = divmod(j, chunks_per_b)
            return b, h

        def gather(j):
            b, h = item_idx(j)
            return pltpu.async_copy(
                wemb_hbm.at[idx_v.at[b, pl.ds(h * _C, _C)]],
                rows[j % _DEPTH], gsem[j % _DEPTH])

        icopy = pltpu.async_copy(
            idx_hbm.at[:, pl.ds(pbase, p_per_w)], idx_v, isem)
        pcopy = pltpu.async_copy(
            pemb_hbm.at[pl.ds(pbase, p_per_w)], pos_v, psem)
        icopy.wait()
        gcur = [gather(j) for j in range(_DEPTH)]
        pcopy.wait()

        ocur = [None] * _DEPTH
        for j in range(n_items):
            buf = j % _DEPTH
            b, h = item_idx(j)
            gcur[buf].wait()
            rbuf = rows[buf]
            prow = h * _C

            @plsc.parallel_loop(0, _C, unroll=4)
            def add_body(r, rbuf=rbuf, prow=prow):
                for c in range(_D // _L):
                    sl = pl.ds(c * _L, _L)
                    plsc.addupdate(rbuf.at[r, sl], pos_v[prow + r, sl])

            ocur[buf] = pltpu.async_copy(
                rbuf, out_hbm.at[b, pl.ds(pbase + prow, _C)], osem[buf])
            nxt = j + 2
            if _DEPTH <= nxt < n_items:
                ocur[nxt % _DEPTH].wait()
                gcur[nxt % _DEPTH] = gather(nxt)
        for buf in range(_DEPTH):
            ocur[buf].wait()

    return k


def kernel(input_ids, word_embeddings, position_embeddings):
    if input_ids.dtype != jnp.int32:
        input_ids = input_ids.astype(jnp.int32)
    return _make_kernel()(input_ids, word_embeddings, position_embeddings)
